# baseline (speedup 1.0000x reference)
import functools

import jax
import jax.numpy as jnp
from jax import lax
from jax.experimental import pallas as pl
from jax.experimental.pallas import tpu as pltpu
from jax.experimental.pallas import tpu_sc as plsc

D_MODEL = 768
BATCH = 4
SEQ = 2048
N_TOKENS = BATCH * SEQ

_info = plsc.get_sparse_core_info()
_NC, _NS = _info.num_cores, _info.num_subcores
_NW = _NC * _NS
_B_PER_W = N_TOKENS // _NW
_W_PER_ROW = SEQ // _B_PER_W
_CHUNK = 32
_NCHUNK = _B_PER_W // _CHUNK
_NBUF = 4


def _sc_gather(table, tokens):
    mesh = plsc.VectorSubcoreMesh(core_axis_name="c", subcore_axis_name="s")

    @functools.partial(
        pl.kernel,
        mesh=mesh,
        out_type=jax.ShapeDtypeStruct((BATCH, SEQ, D_MODEL), jnp.float32),
        scratch_types=[
            pltpu.VMEM((_B_PER_W,), jnp.int32),
            pltpu.VMEM((_NBUF, _CHUNK, D_MODEL), jnp.float32),
            pltpu.SemaphoreType.DMA,
            pltpu.SemaphoreType.DMA,
            pltpu.SemaphoreType.DMA,
        ],
    )
    def k(table_hbm, idx_hbm, out_hbm, idx_v, rows_v, isem, gsem0, wsem0):
        wid = lax.axis_index("s") * _NC + lax.axis_index("c")
        base = wid * _B_PER_W
        row = wid // _W_PER_ROW
        col = (wid % _W_PER_ROW) * _B_PER_W
        gsem = [gsem0] * _NBUF
        wsem = [wsem0] * _NBUF
        pltpu.sync_copy(
            idx_hbm.at[pl.ds(base, _CHUNK)], idx_v.at[pl.ds(0, _CHUNK)]
        )
        rest = pltpu.async_copy(
            idx_hbm.at[pl.ds(base + _CHUNK, _B_PER_W - _CHUNK)],
            idx_v.at[pl.ds(_CHUNK, _B_PER_W - _CHUNK)],
            isem,
        )

        def gather(c):
            return pltpu.async_copy(
                table_hbm.at[idx_v.at[pl.ds(c * _CHUNK, _CHUNK)]],
                rows_v.at[c % _NBUF],
                gsem[c % _NBUF],
            )

        def writeback(c):
            return pltpu.async_copy(
                rows_v.at[c % _NBUF],
                out_hbm.at[row, pl.ds(col + c * _CHUNK, _CHUNK)],
                wsem[c % _NBUF],
            )

        gathers = [None] * _NCHUNK
        writes = [None] * _NCHUNK
        gathers[0] = gather(0)
        rest.wait()
        for c in range(1, _NBUF):
            gathers[c] = gather(c)
        for c in range(_NCHUNK):
            gathers[c].wait()
            writes[c] = writeback(c)
            nxt = c + _NBUF
            if nxt < _NCHUNK:
                writes[c].wait()
                gathers[nxt] = gather(nxt)
        for c in range(_NCHUNK - _NBUF, _NCHUNK):
            writes[c].wait()

    return k(table, tokens)


def kernel(tokens, table):
    if tokens.dtype != jnp.int32:
        tokens = tokens.astype(jnp.int32)
    return _sc_gather(table, tokens.reshape(-1))


# speedup vs baseline: 1.0192x; 1.0192x over previous
import functools

import jax
import jax.numpy as jnp
from jax import lax
from jax.experimental import pallas as pl
from jax.experimental.pallas import tpu as pltpu
from jax.experimental.pallas import tpu_sc as plsc

D_MODEL = 768
BATCH = 4
SEQ = 2048
N_TOKENS = BATCH * SEQ

_info = plsc.get_sparse_core_info()
_NC, _NS = _info.num_cores, _info.num_subcores
_NW = _NC * _NS
_B_PER_W = N_TOKENS // _NW
_W_PER_ROW = SEQ // _B_PER_W
_CHUNK = 32
_NCHUNK = _B_PER_W // _CHUNK
_NBUF = 4


def _sc_gather(table, tokens):
    mesh = plsc.VectorSubcoreMesh(core_axis_name="c", subcore_axis_name="s")

    @functools.partial(
        pl.kernel,
        mesh=mesh,
        out_type=jax.ShapeDtypeStruct((BATCH, SEQ, D_MODEL), jnp.float32),
        scratch_types=[
            pltpu.VMEM((_B_PER_W,), jnp.int32),
            pltpu.VMEM((_NBUF, _CHUNK, D_MODEL), jnp.float32),
            pltpu.SemaphoreType.DMA,
        ] + [pltpu.SemaphoreType.DMA] * (2 * _NBUF),
    )
    def k(table_hbm, idx_hbm, out_hbm, idx_v, rows_v, isem, *sems):
        wid = lax.axis_index("s") * _NC + lax.axis_index("c")
        base = wid * _B_PER_W
        row = wid // _W_PER_ROW
        col = (wid % _W_PER_ROW) * _B_PER_W
        gsem = sems[:_NBUF]
        wsem = sems[_NBUF:]
        pltpu.sync_copy(
            idx_hbm.at[pl.ds(base, _CHUNK)], idx_v.at[pl.ds(0, _CHUNK)]
        )
        rest = pltpu.async_copy(
            idx_hbm.at[pl.ds(base + _CHUNK, _B_PER_W - _CHUNK)],
            idx_v.at[pl.ds(_CHUNK, _B_PER_W - _CHUNK)],
            isem,
        )

        def gather(c):
            return pltpu.async_copy(
                table_hbm.at[idx_v.at[pl.ds(c * _CHUNK, _CHUNK)]],
                rows_v.at[c % _NBUF],
                gsem[c % _NBUF],
            )

        def writeback(c):
            return pltpu.async_copy(
                rows_v.at[c % _NBUF],
                out_hbm.at[row, pl.ds(col + c * _CHUNK, _CHUNK)],
                wsem[c % _NBUF],
            )

        gathers = [None] * _NCHUNK
        writes = [None] * _NCHUNK
        gathers[0] = gather(0)
        rest.wait()
        for c in range(1, _NBUF):
            gathers[c] = gather(c)
        for c in range(_NCHUNK):
            gathers[c].wait()
            writes[c] = writeback(c)
            nxt = c + _NBUF
            if nxt < _NCHUNK:
                writes[c].wait()
                gathers[nxt] = gather(nxt)
        for c in range(_NCHUNK - _NBUF, _NCHUNK):
            writes[c].wait()

    return k(table, tokens)


def kernel(tokens, table):
    if tokens.dtype != jnp.int32:
        tokens = tokens.astype(jnp.int32)
    return _sc_gather(table, tokens.reshape(-1))
